# baseline (speedup 1.0000x reference)
import functools

import jax
import jax.numpy as jnp
from jax import lax
from jax.experimental import pallas as pl
from jax.experimental.pallas import tpu as pltpu
from jax.experimental.pallas import tpu_sc as plsc

NUM_EMB = 1_000_000
D = 32
BATCH = 16384
N_FIELDS = 26
B_TOTAL = BATCH * N_FIELDS

NC = 2
NT = 16
NW = NC * NT
BW_ = BATCH // NW
G = 128
SB = BW_ // G
NG = N_FIELDS * SB
K = 8
NS = 16
TB = 4


def _build():
  mesh = plsc.VectorSubcoreMesh(core_axis_name="c", subcore_axis_name="s")

  @functools.partial(
      pl.kernel,
      mesh=mesh,
      out_type=jax.ShapeDtypeStruct((N_FIELDS, D // 8, BATCH // G, 8, G),
                                    jnp.float32),
      scratch_types=[
          pltpu.VMEM((N_FIELDS, BW_), jnp.int32),
          pltpu.VMEM((NS, G, D), jnp.float32),
          pltpu.VMEM((TB, D // 8, 8, G), jnp.float32),
          pltpu.SemaphoreType.DMA,
          pltpu.SemaphoreType.DMA,
      ],
      compiler_params=pltpu.CompilerParams(
          use_tc_tiling_on_sc=False, needs_layout_passes=False
      ),
  )
  def emb_kernel(table_hbm, idx_hbm, out_hbm, idx_v, rows_v, tbuf, gsem, wsem):
    wid = lax.axis_index("s") * NC + lax.axis_index("c")
    b0 = wid * BW_
    pltpu.sync_copy(idx_hbm.at[:, pl.ds(b0, BW_)], idx_v)

    row_iotas = [lax.iota(jnp.int32, 16) + (h * 16) for h in range(8)]

    def gather_for(g, slot):
      f = g // SB
      sb = lax.rem(g, SB)
      pltpu.async_copy(
          table_hbm.at[idx_v.at[f, pl.ds(sb * G, G)]], rows_v.at[slot], gsem
      )

    for b in range(K):
      gather_for(b, b)

    def outer(o, carry):
      for k in range(K):
        g = o * K + k
        s = lax.rem(g, NS)
        t = lax.rem(g, TB)
        pltpu.make_async_copy(
            table_hbm.at[pl.ds(0, G)], rows_v.at[s], gsem
        ).wait()

        @pl.when(g >= TB)
        def _():
          pltpu.make_async_copy(
              tbuf.at[0], out_hbm.at[0].at[:, 0], wsem
          ).wait()

        def col(c, carry2):
          cvec = jnp.full((16,), 0, jnp.int32) + c
          for h in range(8):
            v = plsc.load_gather(rows_v.at[s], [row_iotas[h], cvec])
            tbuf[t, c // 8, lax.rem(c, 8), pl.ds(h * 16, 16)] = v
          return carry2

        lax.fori_loop(0, D, col, 0)

        f = g // SB
        bbg = wid * SB + lax.rem(g, SB)
        pltpu.async_copy(tbuf.at[t], out_hbm.at[f].at[:, bbg], wsem)

        nxt = g + K

        @pl.when(nxt < NG)
        def _():
          gather_for(nxt, lax.rem(nxt, NS))

      return carry

    lax.fori_loop(0, NG // K, outer, 0)

    for _ in range(TB):
      pltpu.make_async_copy(tbuf.at[0], out_hbm.at[0].at[:, 0], wsem).wait()

  return emb_kernel


def kernel(indices, table):
  idx = indices.T.astype(jnp.int32)
  o = _build()(table, idx)
  return o.transpose(2, 4, 0, 1, 3).reshape(BATCH, N_FIELDS, D)


# speedup vs baseline: 1.0023x; 1.0023x over previous
import functools

import jax
import jax.numpy as jnp
from jax import lax
from jax.experimental import pallas as pl
from jax.experimental.pallas import tpu as pltpu
from jax.experimental.pallas import tpu_sc as plsc

NUM_EMB = 1_000_000
D = 32
BATCH = 16384
N_FIELDS = 26
B_TOTAL = BATCH * N_FIELDS

NC = 2
NT = 16
NW = NC * NT
BW_ = BATCH // NW
G = 128
SB = BW_ // G
NG = N_FIELDS * SB
K = 8
NS = 16
TB = 4


def _build():
  mesh = plsc.VectorSubcoreMesh(core_axis_name="c", subcore_axis_name="s")

  @functools.partial(
      pl.kernel,
      mesh=mesh,
      out_type=jax.ShapeDtypeStruct((N_FIELDS, D // 8, BATCH // G, 8, G),
                                    jnp.float32),
      scratch_types=[
          pltpu.VMEM((N_FIELDS, BW_), jnp.int32),
          pltpu.VMEM((NS, G, D), jnp.float32),
          pltpu.VMEM((TB, D // 8, 8, G), jnp.float32),
          pltpu.SemaphoreType.DMA,
          pltpu.SemaphoreType.DMA,
      ],
      compiler_params=pltpu.CompilerParams(
          use_tc_tiling_on_sc=False, needs_layout_passes=False
      ),
  )
  def emb_kernel(table_hbm, idx_hbm, out_hbm, idx_v, rows_v, tbuf, gsem, wsem):
    wid = lax.axis_index("s") * NC + lax.axis_index("c")
    b0 = wid * BW_
    pltpu.sync_copy(idx_hbm.at[:, pl.ds(b0, BW_)], idx_v)

    row_iotas = [lax.iota(jnp.int32, 16) + (h * 16) for h in range(8)]

    def gather_for(g, slot):
      f = g // SB
      sb = lax.rem(g, SB)
      pltpu.async_copy(
          table_hbm.at[idx_v.at[f, pl.ds(sb * G, G)]], rows_v.at[slot], gsem
      )

    for b in range(K):
      gather_for(b, b)

    cvecs = [jnp.full((16,), c, jnp.int32) for c in range(D)]

    def outer(g, carry):
      s = lax.rem(g, NS)
      t = lax.rem(g, TB)
      pltpu.make_async_copy(
          table_hbm.at[pl.ds(0, G)], rows_v.at[s], gsem
      ).wait()

      @pl.when(g >= TB)
      def _():
        pltpu.make_async_copy(
            tbuf.at[0], out_hbm.at[0].at[:, 0], wsem
        ).wait()

      rows = rows_v.at[s]
      for c in range(D):
        for h in range(8):
          v = plsc.load_gather(rows, [row_iotas[h], cvecs[c]])
          tbuf[t, c // 8, c % 8, pl.ds(h * 16, 16)] = v

      f = g // SB
      bbg = wid * SB + lax.rem(g, SB)
      pltpu.async_copy(tbuf.at[t], out_hbm.at[f].at[:, bbg], wsem)

      nxt = g + K

      @pl.when(nxt < NG)
      def _():
        gather_for(nxt, lax.rem(nxt, NS))

      return carry

    lax.fori_loop(0, NG, outer, 0)

    for _ in range(TB):
      pltpu.make_async_copy(tbuf.at[0], out_hbm.at[0].at[:, 0], wsem).wait()

  return emb_kernel


def kernel(indices, table):
  idx = indices.T.astype(jnp.int32)
  o = _build()(table, idx)
  return o.transpose(2, 4, 0, 1, 3).reshape(BATCH, N_FIELDS, D)


# speedup vs baseline: 1.3435x; 1.3404x over previous
import functools

import jax
import jax.numpy as jnp
from jax import lax
from jax.experimental import pallas as pl
from jax.experimental.pallas import tpu as pltpu
from jax.experimental.pallas import tpu_sc as plsc

NUM_EMB = 1_000_000
D = 32
BATCH = 16384
N_FIELDS = 26
B_TOTAL = BATCH * N_FIELDS

NC = 2
NT = 16
NW = NC * NT
BW_ = BATCH // NW
G = 128
SB = BW_ // G
NG = N_FIELDS * SB
K = 8
NS = 16
TB = 4


def _build():
  mesh = plsc.VectorSubcoreMesh(core_axis_name="c", subcore_axis_name="s")

  @functools.partial(
      pl.kernel,
      mesh=mesh,
      out_type=jax.ShapeDtypeStruct((N_FIELDS, D // 8, BATCH // G, 8, G),
                                    jnp.float32),
      scratch_types=[
          pltpu.VMEM((N_FIELDS, BW_), jnp.int32),
          pltpu.VMEM((NS, G, D), jnp.float32),
          pltpu.VMEM((TB, D // 8, 8, G), jnp.float32),
          pltpu.SemaphoreType.DMA,
          pltpu.SemaphoreType.DMA,
      ],
      compiler_params=pltpu.CompilerParams(
          use_tc_tiling_on_sc=False, needs_layout_passes=False
      ),
  )
  def emb_kernel(table_hbm, idx_hbm, out_hbm, idx_v, rows_v, tbuf, gsem, wsem):
    wid = lax.axis_index("s") * NC + lax.axis_index("c")
    b0 = wid * BW_
    pltpu.sync_copy(idx_hbm.at[:, pl.ds(b0, BW_)], idx_v)

    iotav = lax.iota(jnp.int32, 16)

    def gather_for(g, slot):
      f = g // SB
      sb = lax.rem(g, SB)
      pltpu.async_copy(
          table_hbm.at[idx_v.at[f, pl.ds(sb * G, G)]], rows_v.at[slot], gsem
      )

    for b in range(K):
      gather_for(b, b)

    def outer(g, carry):
      s = lax.rem(g, NS)
      t = lax.rem(g, TB)
      pltpu.make_async_copy(
          table_hbm.at[pl.ds(0, G)], rows_v.at[s], gsem
      ).wait()

      @pl.when(g >= TB)
      def _():
        pltpu.make_async_copy(
            tbuf.at[0], out_hbm.at[0].at[:, 0], wsem
        ).wait()

      rows = rows_v.at[s]
      tb_t = tbuf.at[t]

      def diag(d, carry2):
        cvec = lax.rem(iotav + d, D)
        cb = lax.shift_right_logical(cvec, 3)
        cr = lax.rem(cvec, 8)
        for j0 in range(0, G, 16):
          rvec = iotav + j0
          v = plsc.load_gather(rows, [rvec, cvec])
          plsc.store_scatter(tb_t, [cb, cr, rvec], v)
        return carry2

      lax.fori_loop(0, D, diag, 0)

      f = g // SB
      bbg = wid * SB + lax.rem(g, SB)
      pltpu.async_copy(tbuf.at[t], out_hbm.at[f].at[:, bbg], wsem)

      nxt = g + K

      @pl.when(nxt < NG)
      def _():
        gather_for(nxt, lax.rem(nxt, NS))

      return carry

    lax.fori_loop(0, NG, outer, 0)

    for _ in range(TB):
      pltpu.make_async_copy(tbuf.at[0], out_hbm.at[0].at[:, 0], wsem).wait()

  return emb_kernel


def kernel(indices, table):
  idx = indices.T.astype(jnp.int32)
  o = _build()(table, idx)
  return o.transpose(2, 4, 0, 1, 3).reshape(BATCH, N_FIELDS, D)


# speedup vs baseline: 1.3445x; 1.0007x over previous
import functools

import jax
import jax.numpy as jnp
from jax import lax
from jax.experimental import pallas as pl
from jax.experimental.pallas import tpu as pltpu
from jax.experimental.pallas import tpu_sc as plsc

NUM_EMB = 1_000_000
D = 32
BATCH = 16384
N_FIELDS = 26
B_TOTAL = BATCH * N_FIELDS

NC = 2
NT = 16
NW = NC * NT
BW_ = BATCH // NW
G = 128
SB = BW_ // G
NG = N_FIELDS * SB
K = 8
NS = 16
TB = 4


def _build():
  mesh = plsc.VectorSubcoreMesh(core_axis_name="c", subcore_axis_name="s")

  @functools.partial(
      pl.kernel,
      mesh=mesh,
      out_type=jax.ShapeDtypeStruct((N_FIELDS, D // 8, BATCH // G, 8, G),
                                    jnp.float32),
      scratch_types=[
          pltpu.VMEM((N_FIELDS, BW_), jnp.int32),
          pltpu.VMEM((NS, G, D), jnp.float32),
          pltpu.VMEM((TB, D // 8, 8, G), jnp.float32),
          pltpu.SemaphoreType.DMA,
          pltpu.SemaphoreType.DMA,
      ],
      compiler_params=pltpu.CompilerParams(
          use_tc_tiling_on_sc=False, needs_layout_passes=False
      ),
  )
  def emb_kernel(table_hbm, idx_hbm, out_hbm, idx_v, rows_v, tbuf, gsem, wsem):
    wid = lax.axis_index("s") * NC + lax.axis_index("c")
    b0 = wid * BW_
    pltpu.sync_copy(idx_hbm.at[:, pl.ds(b0, BW_)], idx_v)

    iotav = lax.iota(jnp.int32, 16)

    def gather_for(g, slot):
      f = g // SB
      sb = lax.rem(g, SB)
      pltpu.async_copy(
          table_hbm.at[idx_v.at[f, pl.ds(sb * G, G)]], rows_v.at[slot], gsem
      )

    for b in range(K):
      gather_for(b, b)

    def outer(g, carry):
      s = lax.rem(g, NS)
      t = lax.rem(g, TB)
      pltpu.make_async_copy(
          table_hbm.at[pl.ds(0, G)], rows_v.at[s], gsem
      ).wait()

      nxt0 = g + K

      @pl.when(nxt0 < NG)
      def _():
        gather_for(nxt0, lax.rem(nxt0, NS))

      @pl.when(g >= TB)
      def _():
        pltpu.make_async_copy(
            tbuf.at[0], out_hbm.at[0].at[:, 0], wsem
        ).wait()

      rows = rows_v.at[s]
      tb_t = tbuf.at[t]

      def diag(d, carry2):
        cvec = lax.rem(iotav + d, D)
        cb = lax.shift_right_logical(cvec, 3)
        cr = lax.rem(cvec, 8)
        for j0 in range(0, G, 16):
          rvec = iotav + j0
          v = plsc.load_gather(rows, [rvec, cvec])
          plsc.store_scatter(tb_t, [cb, cr, rvec], v)
        return carry2

      lax.fori_loop(0, D, diag, 0)

      f = g // SB
      bbg = wid * SB + lax.rem(g, SB)
      pltpu.async_copy(tbuf.at[t], out_hbm.at[f].at[:, bbg], wsem)

      return carry

    lax.fori_loop(0, NG, outer, 0)

    for _ in range(TB):
      pltpu.make_async_copy(tbuf.at[0], out_hbm.at[0].at[:, 0], wsem).wait()

  return emb_kernel


def kernel(indices, table):
  idx = indices.T.astype(jnp.int32)
  o = _build()(table, idx)
  return o.transpose(2, 4, 0, 1, 3).reshape(BATCH, N_FIELDS, D)
